# baseline (speedup 1.0000x reference)
import jax
import jax.numpy as jnp
from jax.experimental import pallas as pl
from jax.experimental.pallas import tpu as pltpu

NUM_ANCHORS = 9
NUM_CLASSES = 80
SPATIALS = [4096, 1024, 256, 64, 16]
N_ROWS = sum(SPATIALS) * NUM_ANCHORS
CONF_THR = 0.97
IOU_THR = 0.5
MAX_OUT = 100
NMS_CAP = 4096
MAX_EDGE = 512.0

_SPATIAL_BLK = [512, 512, 256, 64, 16]
_NMS_BLK = 512
_NMS_GRID = NMS_CAP // _NMS_BLK


def _lvl_body(x_ref, out_ref, skey_ref, cat_ref):
    x = x_ref[...]
    sb = x.shape[1]
    t = x.reshape(NUM_ANCHORS, NUM_CLASSES, sb).transpose(2, 0, 1)
    t = t.reshape(sb * NUM_ANCHORS, NUM_CLASSES)
    out_ref[...] = t
    s = jax.nn.sigmoid(t)
    m = jnp.max(s, axis=1)
    idx = jax.lax.broadcasted_iota(jnp.int32, s.shape, 1)
    cat = jnp.min(jnp.where(s == m[:, None], idx, NUM_CLASSES), axis=1)
    skey_ref[0, 0, :] = jnp.where(m >= CONF_THR, m, -1.0)
    cat_ref[0, 0, :] = cat


def _lvl_first(x_ref, out_ref, skey_ref, cat_ref):
    _lvl_body(x_ref, out_ref, skey_ref, cat_ref)


def _lvl_next(x_ref, buf_ref, out_ref, skey_ref, cat_ref):
    del buf_ref
    _lvl_body(x_ref, out_ref, skey_ref, cat_ref)


def _decode_kernel(ax1, ay1, ax2, ay2, dx, dy, dw, dh, cat,
                   rx1, ry1, rx2, ry2, sx1, sy1, sx2, sy2, area):
    aw = ax2[...] - ax1[...]
    ah = ay2[...] - ay1[...]
    acx = ax1[...] + 0.5 * aw
    acy = ay1[...] + 0.5 * ah
    cx = dx[...] * aw + acx
    cy = dy[...] * ah + acy
    w = jnp.exp(jnp.clip(dw[...], -6.0, 6.0)) * aw
    h = jnp.exp(jnp.clip(dh[...], -6.0, 6.0)) * ah
    x1 = cx - w / 2
    y1 = cy - h / 2
    x2 = cx + w / 2
    y2 = cy + h / 2
    rx1[...] = x1
    ry1[...] = y1
    rx2[...] = x2
    ry2[...] = y2
    off = cat[...].astype(jnp.float32) * MAX_EDGE
    bx1 = x1 + off
    by1 = y1 + off
    bx2 = x2 + off
    by2 = y2 + off
    sx1[...] = bx1
    sy1[...] = by1
    sx2[...] = bx2
    sy2[...] = by2
    area[...] = (bx2 - bx1) * (by2 - by1)


def _nms_kernel(x1i, y1i, x2i, y2i, ai,
                x1j, y1j, x2j, y2j, aj, sj, out, acc):
    j = pl.program_id(0)
    i = pl.program_id(1)

    @pl.when(i == 0)
    def _init():
        acc[...] = jnp.zeros_like(acc)

    @pl.when(i <= j)
    def _accum():
        xi = x1i[0].reshape(_NMS_BLK, 1)
        yi = y1i[0].reshape(_NMS_BLK, 1)
        Xi = x2i[0].reshape(_NMS_BLK, 1)
        Yi = y2i[0].reshape(_NMS_BLK, 1)
        Ai = ai[0].reshape(_NMS_BLK, 1)
        ltx = jnp.maximum(xi, x1j[0])
        lty = jnp.maximum(yi, y1j[0])
        rbx = jnp.minimum(Xi, x2j[0])
        rby = jnp.minimum(Yi, y2j[0])
        wx = jnp.maximum(rbx - ltx, 0.0)
        wy = jnp.maximum(rby - lty, 0.0)
        inter = wx * wy
        denom = jnp.maximum(Ai + aj[0] - inter, 1e-9)
        iou = inter / denom
        rg = jax.lax.broadcasted_iota(jnp.int32, iou.shape, 0) + i * _NMS_BLK
        cg = jax.lax.broadcasted_iota(jnp.int32, iou.shape, 1) + j * _NMS_BLK
        masked = jnp.where(rg < cg, iou, 0.0)
        acc[...] = jnp.maximum(acc[...], jnp.max(masked, axis=0, keepdims=True))

    @pl.when(i == j)
    def _finalize():
        s = sj[0]
        keep = acc[...] <= IOU_THR
        out[0] = jnp.where(keep & (s > 0.0), s, 0.0)


def kernel(logits_p3, logits_p4, logits_p5, logits_p6, logits_p7,
           regress_p3, regress_p4, regress_p5, regress_p6, regress_p7,
           anchors):
    logit_lvls = [logits_p3, logits_p4, logits_p5, logits_p6, logits_p7]
    reg_lvls = [regress_p3, regress_p4, regress_p5, regress_p6, regress_p7]

    buf = None
    skeys = []
    catls = []
    row_base = 0
    for lx, S, SB in zip(logit_lvls, SPATIALS, _SPATIAL_BLK):
        x2 = lx.reshape(NUM_ANCHORS * NUM_CLASSES, S)
        nb = S // SB
        rows_blk = SB * NUM_ANCHORS
        base_blk = row_base // rows_blk
        out_shape = [jax.ShapeDtypeStruct((N_ROWS, NUM_CLASSES), jnp.float32),
                     jax.ShapeDtypeStruct((nb, 1, rows_blk), jnp.float32),
                     jax.ShapeDtypeStruct((nb, 1, rows_blk), jnp.int32)]
        out_specs = [
            pl.BlockSpec((rows_blk, NUM_CLASSES),
                         lambda b, _o=base_blk: (_o + b, 0)),
            pl.BlockSpec((1, 1, rows_blk), lambda b: (b, 0, 0)),
            pl.BlockSpec((1, 1, rows_blk), lambda b: (b, 0, 0)),
        ]
        xspec = pl.BlockSpec((NUM_ANCHORS * NUM_CLASSES, SB), lambda b: (0, b))
        if buf is None:
            buf, sk, ct = pl.pallas_call(
                _lvl_first, grid=(nb,),
                in_specs=[xspec],
                out_specs=out_specs, out_shape=out_shape,
            )(x2)
        else:
            buf, sk, ct = pl.pallas_call(
                _lvl_next, grid=(nb,),
                in_specs=[xspec,
                          pl.BlockSpec((8, NUM_CLASSES), lambda b: (0, 0))],
                out_specs=out_specs, out_shape=out_shape,
                input_output_aliases={1: 0},
            )(x2, buf)
        skeys.append(sk.reshape(-1))
        catls.append(ct.reshape(-1))
        row_base += S * NUM_ANCHORS
    lt = buf
    skey = jnp.concatenate(skeys)
    cats = jnp.concatenate(catls)

    rt = jnp.concatenate(
        [x.reshape(NUM_ANCHORS * 4, s).T.reshape(s * NUM_ANCHORS, 4)
         for x, s in zip(reg_lvls, SPATIALS)], axis=0)

    s_s, top = jax.lax.top_k(skey, NMS_CAP)
    cat_top = cats[top]
    reg4 = rt[top]
    anch4 = anchors[top]

    def comp(a, k):
        return a[:, k].reshape(_NMS_GRID, 1, _NMS_BLK)

    cspec = pl.BlockSpec((1, 1, _NMS_BLK), lambda b: (b, 0, 0))
    cshape = jax.ShapeDtypeStruct((_NMS_GRID, 1, _NMS_BLK), jnp.float32)
    rx1, ry1, rx2, ry2, sx1, sy1, sx2, sy2, area = pl.pallas_call(
        _decode_kernel,
        grid=(_NMS_GRID,),
        in_specs=[cspec] * 9,
        out_specs=[cspec] * 9,
        out_shape=[cshape] * 9,
    )(comp(anch4, 0), comp(anch4, 1), comp(anch4, 2), comp(anch4, 3),
      comp(reg4, 0), comp(reg4, 1), comp(reg4, 2), comp(reg4, 3),
      cat_top.astype(jnp.float32).reshape(_NMS_GRID, 1, _NMS_BLK))

    ispec = pl.BlockSpec((1, 1, _NMS_BLK), lambda j, i: (i, 0, 0))
    jspec = pl.BlockSpec((1, 1, _NMS_BLK), lambda j, i: (j, 0, 0))
    sck3 = pl.pallas_call(
        _nms_kernel,
        grid=(_NMS_GRID, _NMS_GRID),
        in_specs=[ispec] * 5 + [jspec] * 6,
        out_specs=pl.BlockSpec((1, 1, _NMS_BLK), lambda j, i: (j, 0, 0)),
        out_shape=jax.ShapeDtypeStruct((_NMS_GRID, 1, _NMS_BLK), jnp.float32),
        scratch_shapes=[pltpu.VMEM((1, _NMS_BLK), jnp.float32)],
    )(sx1, sy1, sx2, sy2, area,
      sx1, sy1, sx2, sy2, area,
      s_s.reshape(_NMS_GRID, 1, _NMS_BLK))
    sc_k = sck3.reshape(NMS_CAP)

    sel_v, sel = jax.lax.top_k(sc_k, MAX_OUT)
    valid = (sel_v > 0.0).astype(jnp.float32)
    raw4 = jnp.stack([rx1.reshape(-1), ry1.reshape(-1),
                      rx2.reshape(-1), ry2.reshape(-1)], axis=1)
    dets = jnp.concatenate([
        jnp.zeros((MAX_OUT, 1), jnp.float32),
        cat_top[sel][:, None].astype(jnp.float32),
        raw4[sel],
        sel_v[:, None],
    ], axis=1) * valid[:, None]
    return dets, lt[None], rt[None]


# speedup vs baseline: 1.2417x; 1.2417x over previous
import jax
import jax.numpy as jnp
from jax.experimental import pallas as pl
from jax.experimental.pallas import tpu as pltpu

NUM_ANCHORS = 9
NUM_CLASSES = 80
SPATIALS = [4096, 1024, 256, 64, 16]
N_ROWS = sum(SPATIALS) * NUM_ANCHORS
CONF_THR = 0.97
IOU_THR = 0.5
MAX_OUT = 100
NMS_CAP = 4096
MAX_EDGE = 512.0

_SPATIAL_BLK = [512, 512, 256, 64, 16]
_NMS_BLK = 512
_NMS_GRID = NMS_CAP // _NMS_BLK


def _score_raw_kernel(x_ref, skey_ref, cat_ref):
    x = x_ref[...]
    sb = x.shape[1]
    s3 = jax.nn.sigmoid(x.reshape(NUM_ANCHORS, NUM_CLASSES, sb))
    m9 = jnp.max(s3, axis=1)
    idx = jax.lax.broadcasted_iota(jnp.int32, s3.shape, 1)
    cat9 = jnp.min(jnp.where(s3 == m9[:, None, :], idx, NUM_CLASSES), axis=1)
    m2 = m9.transpose(1, 0)
    skey_ref[...] = jnp.where(m2 >= CONF_THR, m2, -1.0)
    cat_ref[...] = cat9.transpose(1, 0)


def _decode_kernel(ax1, ay1, ax2, ay2, dx, dy, dw, dh, cat,
                   rx1, ry1, rx2, ry2, sx1, sy1, sx2, sy2, area):
    aw = ax2[...] - ax1[...]
    ah = ay2[...] - ay1[...]
    acx = ax1[...] + 0.5 * aw
    acy = ay1[...] + 0.5 * ah
    cx = dx[...] * aw + acx
    cy = dy[...] * ah + acy
    w = jnp.exp(jnp.clip(dw[...], -6.0, 6.0)) * aw
    h = jnp.exp(jnp.clip(dh[...], -6.0, 6.0)) * ah
    x1 = cx - w / 2
    y1 = cy - h / 2
    x2 = cx + w / 2
    y2 = cy + h / 2
    rx1[...] = x1
    ry1[...] = y1
    rx2[...] = x2
    ry2[...] = y2
    off = cat[...].astype(jnp.float32) * MAX_EDGE
    bx1 = x1 + off
    by1 = y1 + off
    bx2 = x2 + off
    by2 = y2 + off
    sx1[...] = bx1
    sy1[...] = by1
    sx2[...] = bx2
    sy2[...] = by2
    area[...] = (bx2 - bx1) * (by2 - by1)


def _nms_kernel(x1i, y1i, x2i, y2i, ai,
                x1j, y1j, x2j, y2j, aj, sj, out, acc):
    j = pl.program_id(0)
    i = pl.program_id(1)

    @pl.when(i == 0)
    def _init():
        acc[...] = jnp.zeros_like(acc)

    @pl.when(i <= j)
    def _accum():
        xi = x1i[0].reshape(_NMS_BLK, 1)
        yi = y1i[0].reshape(_NMS_BLK, 1)
        Xi = x2i[0].reshape(_NMS_BLK, 1)
        Yi = y2i[0].reshape(_NMS_BLK, 1)
        Ai = ai[0].reshape(_NMS_BLK, 1)
        ltx = jnp.maximum(xi, x1j[0])
        lty = jnp.maximum(yi, y1j[0])
        rbx = jnp.minimum(Xi, x2j[0])
        rby = jnp.minimum(Yi, y2j[0])
        wx = jnp.maximum(rbx - ltx, 0.0)
        wy = jnp.maximum(rby - lty, 0.0)
        inter = wx * wy
        denom = jnp.maximum(Ai + aj[0] - inter, 1e-9)
        iou = inter / denom
        rg = jax.lax.broadcasted_iota(jnp.int32, iou.shape, 0) + i * _NMS_BLK
        cg = jax.lax.broadcasted_iota(jnp.int32, iou.shape, 1) + j * _NMS_BLK
        masked = jnp.where(rg < cg, iou, 0.0)
        acc[...] = jnp.maximum(acc[...], jnp.max(masked, axis=0, keepdims=True))

    @pl.when(i == j)
    def _finalize():
        s = sj[0]
        keep = acc[...] <= IOU_THR
        out[0] = jnp.where(keep & (s > 0.0), s, 0.0)


def kernel(logits_p3, logits_p4, logits_p5, logits_p6, logits_p7,
           regress_p3, regress_p4, regress_p5, regress_p6, regress_p7,
           anchors):
    logit_lvls = [logits_p3, logits_p4, logits_p5, logits_p6, logits_p7]
    reg_lvls = [regress_p3, regress_p4, regress_p5, regress_p6, regress_p7]
    lt = jnp.concatenate(
        [x.reshape(NUM_ANCHORS * NUM_CLASSES, s).T.reshape(s * NUM_ANCHORS, NUM_CLASSES)
         for x, s in zip(logit_lvls, SPATIALS)], axis=0)
    rt = jnp.concatenate(
        [x.reshape(NUM_ANCHORS * 4, s).T.reshape(s * NUM_ANCHORS, 4)
         for x, s in zip(reg_lvls, SPATIALS)], axis=0)

    skeys = []
    catls = []
    for lx, S, SB in zip(logit_lvls, SPATIALS, _SPATIAL_BLK):
        x2 = lx.reshape(NUM_ANCHORS * NUM_CLASSES, S)
        nb = S // SB
        sk, ct = pl.pallas_call(
            _score_raw_kernel,
            grid=(nb,),
            in_specs=[pl.BlockSpec((NUM_ANCHORS * NUM_CLASSES, SB),
                                   lambda b: (0, b))],
            out_specs=[pl.BlockSpec((SB, NUM_ANCHORS), lambda b: (b, 0)),
                       pl.BlockSpec((SB, NUM_ANCHORS), lambda b: (b, 0))],
            out_shape=[jax.ShapeDtypeStruct((S, NUM_ANCHORS), jnp.float32),
                       jax.ShapeDtypeStruct((S, NUM_ANCHORS), jnp.int32)],
        )(x2)
        skeys.append(sk.reshape(-1))
        catls.append(ct.reshape(-1))
    skey = jnp.concatenate(skeys)
    cats = jnp.concatenate(catls)

    s_s, top = jax.lax.top_k(skey, NMS_CAP)
    cat_top = cats[top]
    reg4 = rt[top]
    anch4 = anchors[top]

    def comp(a, k):
        return a[:, k].reshape(_NMS_GRID, 1, _NMS_BLK)

    cspec = pl.BlockSpec((1, 1, _NMS_BLK), lambda b: (b, 0, 0))
    cshape = jax.ShapeDtypeStruct((_NMS_GRID, 1, _NMS_BLK), jnp.float32)
    rx1, ry1, rx2, ry2, sx1, sy1, sx2, sy2, area = pl.pallas_call(
        _decode_kernel,
        grid=(_NMS_GRID,),
        in_specs=[cspec] * 9,
        out_specs=[cspec] * 9,
        out_shape=[cshape] * 9,
    )(comp(anch4, 0), comp(anch4, 1), comp(anch4, 2), comp(anch4, 3),
      comp(reg4, 0), comp(reg4, 1), comp(reg4, 2), comp(reg4, 3),
      cat_top.astype(jnp.float32).reshape(_NMS_GRID, 1, _NMS_BLK))

    ispec = pl.BlockSpec((1, 1, _NMS_BLK), lambda j, i: (i, 0, 0))
    jspec = pl.BlockSpec((1, 1, _NMS_BLK), lambda j, i: (j, 0, 0))
    sck3 = pl.pallas_call(
        _nms_kernel,
        grid=(_NMS_GRID, _NMS_GRID),
        in_specs=[ispec] * 5 + [jspec] * 6,
        out_specs=pl.BlockSpec((1, 1, _NMS_BLK), lambda j, i: (j, 0, 0)),
        out_shape=jax.ShapeDtypeStruct((_NMS_GRID, 1, _NMS_BLK), jnp.float32),
        scratch_shapes=[pltpu.VMEM((1, _NMS_BLK), jnp.float32)],
    )(sx1, sy1, sx2, sy2, area,
      sx1, sy1, sx2, sy2, area,
      s_s.reshape(_NMS_GRID, 1, _NMS_BLK))
    sc_k = sck3.reshape(NMS_CAP)

    sel_v, sel = jax.lax.top_k(sc_k, MAX_OUT)
    valid = (sel_v > 0.0).astype(jnp.float32)
    raw4 = jnp.stack([rx1.reshape(-1), ry1.reshape(-1),
                      rx2.reshape(-1), ry2.reshape(-1)], axis=1)
    dets = jnp.concatenate([
        jnp.zeros((MAX_OUT, 1), jnp.float32),
        cat_top[sel][:, None].astype(jnp.float32),
        raw4[sel],
        sel_v[:, None],
    ], axis=1) * valid[:, None]
    return dets, lt[None], rt[None]


# speedup vs baseline: 1.2663x; 1.0197x over previous
import jax
import jax.numpy as jnp
from jax.experimental import pallas as pl
from jax.experimental.pallas import tpu as pltpu

NUM_ANCHORS = 9
NUM_CLASSES = 80
SPATIALS = [4096, 1024, 256, 64, 16]
N_ROWS = sum(SPATIALS) * NUM_ANCHORS
CONF_THR = 0.97
IOU_THR = 0.5
MAX_OUT = 100
NMS_CAP = 4096
MAX_EDGE = 512.0

_SPATIAL_BLK = [512, 512, 256, 64, 16]
_NMS_BLK = 512
_NMS_GRID = NMS_CAP // _NMS_BLK


def _score_raw_kernel(x_ref, skey_ref, cat_ref):
    x = x_ref[...]
    sb = x.shape[1]
    s3 = jax.nn.sigmoid(x.reshape(NUM_ANCHORS, NUM_CLASSES, sb))
    m9 = jnp.max(s3, axis=1)
    idx = jax.lax.broadcasted_iota(jnp.int32, s3.shape, 1)
    cat9 = jnp.min(jnp.where(s3 == m9[:, None, :], idx, NUM_CLASSES), axis=1)
    m2 = m9.transpose(1, 0)
    skey_ref[...] = jnp.where(m2 >= CONF_THR, m2, -1.0)
    cat_ref[...] = cat9.transpose(1, 0)


def _decode_tile(ax1, ay1, ax2, ay2, dx, dy, dw, dh, catf):
    aw = ax2 - ax1
    ah = ay2 - ay1
    acx = ax1 + 0.5 * aw
    acy = ay1 + 0.5 * ah
    cx = dx * aw + acx
    cy = dy * ah + acy
    w = jnp.exp(jnp.clip(dw, -6.0, 6.0)) * aw
    h = jnp.exp(jnp.clip(dh, -6.0, 6.0)) * ah
    x1 = cx - w / 2
    y1 = cy - h / 2
    x2 = cx + w / 2
    y2 = cy + h / 2
    off = catf * MAX_EDGE
    bx1 = x1 + off
    by1 = y1 + off
    bx2 = x2 + off
    by2 = y2 + off
    area = (bx2 - bx1) * (by2 - by1)
    return (x1, y1, x2, y2), (bx1, by1, bx2, by2, area)


def _decnms_kernel(a1i, a2i, a3i, a4i, d1i, d2i, d3i, d4i, ci,
                   a1j, a2j, a3j, a4j, d1j, d2j, d3j, d4j, cj, sj,
                   out, r1, r2, r3, r4, acc):
    j = pl.program_id(0)
    i = pl.program_id(1)

    def dec_j():
        return _decode_tile(a1j[0], a2j[0], a3j[0], a4j[0],
                            d1j[0], d2j[0], d3j[0], d4j[0], cj[0])

    def iou_tile():
        _, (x1i, y1i, x2i, y2i, ai) = _decode_tile(
            a1i[0], a2i[0], a3i[0], a4i[0],
            d1i[0], d2i[0], d3i[0], d4i[0], ci[0])
        _, (x1j, y1j, x2j, y2j, aj) = dec_j()
        xi = x1i.reshape(_NMS_BLK, 1)
        yi = y1i.reshape(_NMS_BLK, 1)
        Xi = x2i.reshape(_NMS_BLK, 1)
        Yi = y2i.reshape(_NMS_BLK, 1)
        Ai = ai.reshape(_NMS_BLK, 1)
        ltx = jnp.maximum(xi, x1j)
        lty = jnp.maximum(yi, y1j)
        rbx = jnp.minimum(Xi, x2j)
        rby = jnp.minimum(Yi, y2j)
        wx = jnp.maximum(rbx - ltx, 0.0)
        wy = jnp.maximum(rby - lty, 0.0)
        inter = wx * wy
        denom = jnp.maximum(Ai + aj - inter, 1e-9)
        return inter / denom

    @pl.when(i == 0)
    def _init():
        acc[...] = jnp.zeros_like(acc)
        (rx1, ry1, rx2, ry2), _ = dec_j()
        r1[0] = rx1
        r2[0] = ry1
        r3[0] = rx2
        r4[0] = ry2

    @pl.when(i < j)
    def _off_diag():
        iou = iou_tile()
        acc[...] = jnp.maximum(acc[...], jnp.max(iou, axis=0, keepdims=True))

    @pl.when(i == j)
    def _diag():
        iou = iou_tile()
        rg = jax.lax.broadcasted_iota(jnp.int32, iou.shape, 0)
        cg = jax.lax.broadcasted_iota(jnp.int32, iou.shape, 1)
        masked = jnp.where(rg < cg, iou, 0.0)
        acc2 = jnp.maximum(acc[...], jnp.max(masked, axis=0, keepdims=True))
        s = sj[0]
        out[0] = jnp.where((acc2 <= IOU_THR) & (s > 0.0), s, 0.0)


def kernel(logits_p3, logits_p4, logits_p5, logits_p6, logits_p7,
           regress_p3, regress_p4, regress_p5, regress_p6, regress_p7,
           anchors):
    logit_lvls = [logits_p3, logits_p4, logits_p5, logits_p6, logits_p7]
    reg_lvls = [regress_p3, regress_p4, regress_p5, regress_p6, regress_p7]
    lt = jnp.concatenate(
        [x.reshape(NUM_ANCHORS * NUM_CLASSES, s).T.reshape(s * NUM_ANCHORS, NUM_CLASSES)
         for x, s in zip(logit_lvls, SPATIALS)], axis=0)
    rt = jnp.concatenate(
        [x.reshape(NUM_ANCHORS * 4, s).T.reshape(s * NUM_ANCHORS, 4)
         for x, s in zip(reg_lvls, SPATIALS)], axis=0)

    skeys = []
    catls = []
    for lx, S, SB in zip(logit_lvls, SPATIALS, _SPATIAL_BLK):
        x2 = lx.reshape(NUM_ANCHORS * NUM_CLASSES, S)
        nb = S // SB
        sk, ct = pl.pallas_call(
            _score_raw_kernel,
            grid=(nb,),
            in_specs=[pl.BlockSpec((NUM_ANCHORS * NUM_CLASSES, SB),
                                   lambda b: (0, b))],
            out_specs=[pl.BlockSpec((SB, NUM_ANCHORS), lambda b: (b, 0)),
                       pl.BlockSpec((SB, NUM_ANCHORS), lambda b: (b, 0))],
            out_shape=[jax.ShapeDtypeStruct((S, NUM_ANCHORS), jnp.float32),
                       jax.ShapeDtypeStruct((S, NUM_ANCHORS), jnp.int32)],
        )(x2)
        skeys.append(sk.reshape(-1))
        catls.append(ct.reshape(-1))
    skey = jnp.concatenate(skeys)
    cats = jnp.concatenate(catls)

    s_s, top = jax.lax.top_k(skey, NMS_CAP)
    cat_top = cats[top]
    reg4 = rt[top]
    anch4 = anchors[top]

    def comp(a, k):
        return a[:, k].reshape(_NMS_GRID, 1, _NMS_BLK)

    comps = [comp(anch4, 0), comp(anch4, 1), comp(anch4, 2), comp(anch4, 3),
             comp(reg4, 0), comp(reg4, 1), comp(reg4, 2), comp(reg4, 3),
             cat_top.astype(jnp.float32).reshape(_NMS_GRID, 1, _NMS_BLK)]
    ispec = pl.BlockSpec((1, 1, _NMS_BLK), lambda j, i: (i, 0, 0))
    jspec = pl.BlockSpec((1, 1, _NMS_BLK), lambda j, i: (j, 0, 0))
    oshape = jax.ShapeDtypeStruct((_NMS_GRID, 1, _NMS_BLK), jnp.float32)
    sck3, rx1, ry1, rx2, ry2 = pl.pallas_call(
        _decnms_kernel,
        grid=(_NMS_GRID, _NMS_GRID),
        in_specs=[ispec] * 9 + [jspec] * 10,
        out_specs=[pl.BlockSpec((1, 1, _NMS_BLK), lambda j, i: (j, 0, 0))] * 5,
        out_shape=[oshape] * 5,
        scratch_shapes=[pltpu.VMEM((1, _NMS_BLK), jnp.float32)],
    )(*comps, *comps, s_s.reshape(_NMS_GRID, 1, _NMS_BLK))
    sc_k = sck3.reshape(NMS_CAP)

    sel_v, sel = jax.lax.top_k(sc_k, MAX_OUT)
    valid = (sel_v > 0.0).astype(jnp.float32)
    raw4 = jnp.stack([rx1.reshape(-1), ry1.reshape(-1),
                      rx2.reshape(-1), ry2.reshape(-1)], axis=1)
    dets = jnp.concatenate([
        jnp.zeros((MAX_OUT, 1), jnp.float32),
        cat_top[sel][:, None].astype(jnp.float32),
        raw4[sel],
        sel_v[:, None],
    ], axis=1) * valid[:, None]
    return dets, lt[None], rt[None]


# speedup vs baseline: 1.4383x; 1.1359x over previous
import jax
import jax.numpy as jnp
from jax.experimental import pallas as pl
from jax.experimental.pallas import tpu as pltpu

NUM_ANCHORS = 9
NUM_CLASSES = 80
SPATIALS = [4096, 1024, 256, 64, 16]
N_ROWS = sum(SPATIALS) * NUM_ANCHORS
CONF_THR = 0.97
IOU_THR = 0.5
MAX_OUT = 100
NMS_CAP = 4096
MAX_EDGE = 512.0

_SPATIAL_BLK = [512, 512, 256, 64, 16]
_NMS_BLK = 512
_NMS_GRID = NMS_CAP // _NMS_BLK


def _score_raw_kernel(x_ref, skey_ref, cat_ref):
    x = x_ref[...]
    sb = x.shape[1]
    s3 = jax.nn.sigmoid(x.reshape(NUM_ANCHORS, NUM_CLASSES, sb))
    m9 = jnp.max(s3, axis=1)
    idx = jax.lax.broadcasted_iota(jnp.int32, s3.shape, 1)
    cat9 = jnp.min(jnp.where(s3 == m9[:, None, :], idx, NUM_CLASSES), axis=1)
    m2 = m9.transpose(1, 0)
    skey_ref[...] = jnp.where(m2 >= CONF_THR, m2, -1.0)
    cat_ref[...] = cat9.transpose(1, 0)


def _decode_tile(ax1, ay1, ax2, ay2, dx, dy, dw, dh, catf):
    aw = ax2 - ax1
    ah = ay2 - ay1
    acx = ax1 + 0.5 * aw
    acy = ay1 + 0.5 * ah
    cx = dx * aw + acx
    cy = dy * ah + acy
    w = jnp.exp(jnp.clip(dw, -6.0, 6.0)) * aw
    h = jnp.exp(jnp.clip(dh, -6.0, 6.0)) * ah
    x1 = cx - w / 2
    y1 = cy - h / 2
    x2 = cx + w / 2
    y2 = cy + h / 2
    off = catf * MAX_EDGE
    bx1 = x1 + off
    by1 = y1 + off
    bx2 = x2 + off
    by2 = y2 + off
    area = (bx2 - bx1) * (by2 - by1)
    return (x1, y1, x2, y2), (bx1, by1, bx2, by2, area)


def _decnms_kernel(a1i, a2i, a3i, a4i, d1i, d2i, d3i, d4i, ci,
                   a1j, a2j, a3j, a4j, d1j, d2j, d3j, d4j, cj, sj,
                   out, r1, r2, r3, r4, acc):
    j = pl.program_id(0)
    i = pl.program_id(1)

    def dec_j():
        return _decode_tile(a1j[0], a2j[0], a3j[0], a4j[0],
                            d1j[0], d2j[0], d3j[0], d4j[0], cj[0])

    def iou_tile():
        _, (x1i, y1i, x2i, y2i, ai) = _decode_tile(
            a1i[0], a2i[0], a3i[0], a4i[0],
            d1i[0], d2i[0], d3i[0], d4i[0], ci[0])
        _, (x1j, y1j, x2j, y2j, aj) = dec_j()
        xi = x1i.reshape(_NMS_BLK, 1)
        yi = y1i.reshape(_NMS_BLK, 1)
        Xi = x2i.reshape(_NMS_BLK, 1)
        Yi = y2i.reshape(_NMS_BLK, 1)
        Ai = ai.reshape(_NMS_BLK, 1)
        ltx = jnp.maximum(xi, x1j)
        lty = jnp.maximum(yi, y1j)
        rbx = jnp.minimum(Xi, x2j)
        rby = jnp.minimum(Yi, y2j)
        wx = jnp.maximum(rbx - ltx, 0.0)
        wy = jnp.maximum(rby - lty, 0.0)
        inter = wx * wy
        denom = jnp.maximum(Ai + aj - inter, 1e-9)
        return inter / denom

    @pl.when(i == 0)
    def _init():
        acc[...] = jnp.zeros_like(acc)
        (rx1, ry1, rx2, ry2), _ = dec_j()
        r1[0] = rx1
        r2[0] = ry1
        r3[0] = rx2
        r4[0] = ry2

    @pl.when(i < j)
    def _off_diag():
        iou = iou_tile()
        acc[...] = jnp.maximum(acc[...], jnp.max(iou, axis=0, keepdims=True))

    @pl.when(i == j)
    def _diag():
        iou = iou_tile()
        rg = jax.lax.broadcasted_iota(jnp.int32, iou.shape, 0)
        cg = jax.lax.broadcasted_iota(jnp.int32, iou.shape, 1)
        masked = jnp.where(rg < cg, iou, 0.0)
        acc2 = jnp.maximum(acc[...], jnp.max(masked, axis=0, keepdims=True))
        s = sj[0]
        out[0] = jnp.where((acc2 <= IOU_THR) & (s > 0.0), s, 0.0)


def kernel(logits_p3, logits_p4, logits_p5, logits_p6, logits_p7,
           regress_p3, regress_p4, regress_p5, regress_p6, regress_p7,
           anchors):
    logit_lvls = [logits_p3, logits_p4, logits_p5, logits_p6, logits_p7]
    reg_lvls = [regress_p3, regress_p4, regress_p5, regress_p6, regress_p7]
    lt = jnp.concatenate(
        [x.reshape(NUM_ANCHORS * NUM_CLASSES, s).T.reshape(s * NUM_ANCHORS, NUM_CLASSES)
         for x, s in zip(logit_lvls, SPATIALS)], axis=0)
    rt = jnp.concatenate(
        [x.reshape(NUM_ANCHORS * 4, s).T.reshape(s * NUM_ANCHORS, 4)
         for x, s in zip(reg_lvls, SPATIALS)], axis=0)

    skeys = []
    catls = []
    for lx, S, SB in zip(logit_lvls, SPATIALS, _SPATIAL_BLK):
        x2 = lx.reshape(NUM_ANCHORS * NUM_CLASSES, S)
        nb = S // SB
        sk, ct = pl.pallas_call(
            _score_raw_kernel,
            grid=(nb,),
            in_specs=[pl.BlockSpec((NUM_ANCHORS * NUM_CLASSES, SB),
                                   lambda b: (0, b))],
            out_specs=[pl.BlockSpec((SB, NUM_ANCHORS), lambda b: (b, 0)),
                       pl.BlockSpec((SB, NUM_ANCHORS), lambda b: (b, 0))],
            out_shape=[jax.ShapeDtypeStruct((S, NUM_ANCHORS), jnp.float32),
                       jax.ShapeDtypeStruct((S, NUM_ANCHORS), jnp.int32)],
        )(x2)
        skeys.append(sk.reshape(-1))
        catls.append(ct.reshape(-1))
    skey = jnp.concatenate(skeys)
    cats = jnp.concatenate(catls)

    s_s, top = jax.lax.top_k(skey, NMS_CAP)
    packed = jnp.concatenate(
        [rt, anchors, cats.astype(jnp.float32)[:, None]], axis=1)
    gath = packed[top]
    reg4 = gath[:, 0:4]
    anch4 = gath[:, 4:8]
    catf = gath[:, 8]

    def comp(a, k):
        return a[:, k].reshape(_NMS_GRID, 1, _NMS_BLK)

    comps = [comp(anch4, 0), comp(anch4, 1), comp(anch4, 2), comp(anch4, 3),
             comp(reg4, 0), comp(reg4, 1), comp(reg4, 2), comp(reg4, 3),
             catf.reshape(_NMS_GRID, 1, _NMS_BLK)]
    ispec = pl.BlockSpec((1, 1, _NMS_BLK), lambda j, i: (i, 0, 0))
    jspec = pl.BlockSpec((1, 1, _NMS_BLK), lambda j, i: (j, 0, 0))
    oshape = jax.ShapeDtypeStruct((_NMS_GRID, 1, _NMS_BLK), jnp.float32)
    sck3, rx1, ry1, rx2, ry2 = pl.pallas_call(
        _decnms_kernel,
        grid=(_NMS_GRID, _NMS_GRID),
        in_specs=[ispec] * 9 + [jspec] * 10,
        out_specs=[pl.BlockSpec((1, 1, _NMS_BLK), lambda j, i: (j, 0, 0))] * 5,
        out_shape=[oshape] * 5,
        scratch_shapes=[pltpu.VMEM((1, _NMS_BLK), jnp.float32)],
    )(*comps, *comps, s_s.reshape(_NMS_GRID, 1, _NMS_BLK))
    sc_k = sck3.reshape(NMS_CAP)

    sel_v, sel = jax.lax.top_k(sc_k, MAX_OUT)
    valid = (sel_v > 0.0).astype(jnp.float32)
    raw4 = jnp.stack([rx1.reshape(-1), ry1.reshape(-1),
                      rx2.reshape(-1), ry2.reshape(-1)], axis=1)
    dets = jnp.concatenate([
        jnp.zeros((MAX_OUT, 1), jnp.float32),
        catf[sel][:, None],
        raw4[sel],
        sel_v[:, None],
    ], axis=1) * valid[:, None]
    return dets, lt[None], rt[None]
